# Initial kernel scaffold; baseline (speedup 1.0000x reference)
import jax
import jax.numpy as jnp
from jax.experimental import pallas as pl


def kernel(x, edge_index, Wn0, Wr0, b0, Wn1, Wr1, b1, Wn2, Wr2, b2):
    raise NotImplementedError("write your pallas kernel here")


# speedup vs baseline: 9.3143x; 9.3143x over previous
import functools

import jax
import jax.numpy as jnp
from jax import lax
from jax.experimental import pallas as pl
from jax.experimental.pallas import tpu as pltpu
from jax.experimental.pallas import tpu_sc as plsc

NC = 2
NS = 16
NW = NC * NS

CHUNK = 80
KB = 8


def _sc_aggregate(n, d, k, with_deg):
    rpt = (n // NS + CHUNK - 1) // CHUNK * CHUNK
    npad = rpt * NS
    mesh = plsc.VectorSubcoreMesh(core_axis_name="c", subcore_axis_name="s",
                                  num_cores=NC, num_subcores=NS)

    out_type = jax.ShapeDtypeStruct((NC, npad, d), jnp.float32)
    scratch = [
        pltpu.VMEM((KB, CHUNK), jnp.int32),
        pltpu.VMEM((KB, CHUNK), jnp.int32),
        pltpu.VMEM((CHUNK, d), jnp.float32),
        pltpu.VMEM_SHARED((npad, d), jnp.float32),
        pltpu.SemaphoreType.DMA,
    ]

    def body(h, src, dst, accp, sidx, didx, rows, acc_sh, sem):
        c = lax.axis_index("c")
        s = lax.axis_index("s")
        wid = c * NS + s

        zv = jnp.zeros((16,), jnp.float32)
        ov = jnp.ones((16,), jnp.float32)

        def fill(val):
            def frow(i, carry):
                for t in range(d // 16):
                    rows[i, pl.ds(t * 16, 16)] = val
                return carry
            lax.fori_loop(0, CHUNK, frow, 0)

        fill(zv)

        for j in range(rpt // CHUNK):
            sl = pl.ds(s * rpt + j * CHUNK, CHUNK)
            pltpu.sync_copy(rows, acc_sh.at[sl])
        if with_deg:
            fill(ov)
        plsc.subcore_barrier()

        def blk(bi, carry):
            if not with_deg:
                pltpu.sync_copy(src.at[wid, pl.ds(bi * KB, KB)], sidx)
            pltpu.sync_copy(dst.at[wid, pl.ds(bi * KB, KB)], didx)
            for j in range(KB):
                if not with_deg:
                    pltpu.async_copy(h.at[sidx.at[j]], rows, sem).wait()
                pltpu.sync_copy(rows, acc_sh.at[didx.at[j]], add=True)
            return carry

        lax.fori_loop(0, k // KB, blk, 0)
        plsc.subcore_barrier()

        sl = pl.ds(s * rpt, rpt)
        pltpu.sync_copy(acc_sh.at[sl], accp.at[c, sl])

    return pl.kernel(
        body,
        out_type=out_type,
        mesh=mesh,
        scratch_types=scratch,
    )


def _tc_layer_body(relu, accp_ref, degp_ref, h_ref, wn_ref, wr_ref, b_ref,
                   out_ref):
    hh = h_ref[...]
    agg = accp_ref[0] + accp_ref[1] + hh
    deg = degp_ref[0, :, 0:1] + degp_ref[1, :, 0:1] + 1.0
    mean = agg / deg
    o = (jnp.dot(mean, wn_ref[...], preferred_element_type=jnp.float32)
         + jnp.dot(hh, wr_ref[...], preferred_element_type=jnp.float32)
         + b_ref[...])
    if relu:
        o = jnp.maximum(o, 0.0)
    out_ref[...] = o


def _tc_layer(accp, degp, h, wn, wr, b, relu, block):
    n, d = h.shape
    grid = n // block
    return pl.pallas_call(
        functools.partial(_tc_layer_body, relu),
        grid=(grid,),
        in_specs=[
            pl.BlockSpec((NC, block, d), lambda i: (0, i, 0)),
            pl.BlockSpec((NC, block, 1), lambda i: (0, i, 0)),
            pl.BlockSpec((block, d), lambda i: (i, 0)),
            pl.BlockSpec((d, d), lambda i: (0, 0)),
            pl.BlockSpec((d, d), lambda i: (0, 0)),
            pl.BlockSpec((1, d), lambda i: (0, 0)),
        ],
        out_specs=pl.BlockSpec((block, d), lambda i: (i, 0)),
        out_shape=jax.ShapeDtypeStruct((n, d), jnp.float32),
    )(accp, degp, h, wn, wr, b)


def kernel(x, edge_index, Wn0, Wr0, b0, Wn1, Wr1, b1, Wn2, Wr2, b2):
    n, d = x.shape
    e = edge_index.shape[1]
    epw = e // NW
    k = -(-epw // CHUNK)
    assert epw * NW == e and n % NS == 0

    rpt = (n // NS + CHUNK - 1) // CHUNK * CHUNK
    npad = rpt * NS
    src2 = edge_index[0].reshape(NW, epw)
    dst2 = edge_index[1].reshape(NW, epw)
    kb = -(-k // KB) * KB
    pad = kb * CHUNK - epw
    if pad:
        ap = jnp.arange(pad, dtype=jnp.int32)
        pad_src = jnp.broadcast_to((ap * 37) % n, (NW, pad))
        pad_dst = jnp.broadcast_to(n + ap % (npad - n), (NW, pad))
        src2 = jnp.concatenate([src2, pad_src], axis=1)
        dst2 = jnp.concatenate([dst2, pad_dst], axis=1)
    k = kb
    src = src2.reshape(NW, k, CHUNK)
    dst = dst2.reshape(NW, k, CHUNK)

    sc = _sc_aggregate(n, d, k, with_deg=False)
    scdeg = _sc_aggregate(n, d, k, with_deg=True)

    degp = scdeg(x, src, dst)[:, :, 0:1]
    accp = sc(x, src, dst)
    h1 = _tc_layer(accp, degp, x, Wn0, Wr0, b0.reshape(1, d), True, 2000)
    accp1 = sc(h1, src, dst)
    h2 = _tc_layer(accp1, degp, h1, Wn1, Wr1, b1.reshape(1, d), True, 2000)
    accp2 = sc(h2, src, dst)
    return _tc_layer(accp2, degp, h2, Wn2, Wr2, b2.reshape(1, d), False, 2000)


# speedup vs baseline: 14.4505x; 1.5514x over previous
import functools

import jax
import jax.numpy as jnp
from jax import lax
from jax.experimental import pallas as pl
from jax.experimental.pallas import tpu as pltpu
from jax.experimental.pallas import tpu_sc as plsc

NC = 2
NS = 16
NW = NC * NS

CHUNK = 128
KB = 8


def _sc_aggregate(n, d, k, with_deg):
    rpt = (n // NS + CHUNK - 1) // CHUNK * CHUNK
    npad = rpt * NS
    mesh = plsc.VectorSubcoreMesh(core_axis_name="c", subcore_axis_name="s",
                                  num_cores=NC, num_subcores=NS)

    out_type = jax.ShapeDtypeStruct((NC, npad, d), jnp.float32)
    scratch = [
        pltpu.VMEM((KB, CHUNK), jnp.int32),
        pltpu.VMEM((KB, CHUNK), jnp.int32),
        pltpu.VMEM((CHUNK, d), jnp.float32),
        pltpu.VMEM((CHUNK, d), jnp.float32),
        pltpu.VMEM_SHARED((npad, d), jnp.float32),
        pltpu.SemaphoreType.DMA,
        pltpu.SemaphoreType.DMA,
    ]

    def body(h, src, dst, accp, sidx, didx, rows0, rows1, acc_sh, s0, s1):
        c = lax.axis_index("c")
        s = lax.axis_index("s")
        wid = c * NS + s
        rows = (rows0, rows1)
        sems = (s0, s1)

        zv = jnp.zeros((16,), jnp.float32)
        ov = jnp.ones((16,), jnp.float32)

        def fill(val):
            def frow(i, carry):
                for t in range(d // 16):
                    rows0[i, pl.ds(t * 16, 16)] = val
                return carry
            lax.fori_loop(0, CHUNK, frow, 0)

        fill(zv)

        for j in range(rpt // CHUNK):
            sl = pl.ds(s * rpt + j * CHUNK, CHUNK)
            pltpu.sync_copy(rows0, acc_sh.at[sl])
        if with_deg:
            fill(ov)
        plsc.subcore_barrier()

        if with_deg:
            def dblk(bi, carry):
                pltpu.sync_copy(dst.at[wid, pl.ds(bi * KB, KB)], didx)
                for j in range(KB):
                    pltpu.sync_copy(rows0, acc_sh.at[didx.at[j]], add=True)
                return carry

            lax.fori_loop(0, k // KB, dblk, 0)
        else:
            def blk(bi, carry):
                pltpu.sync_copy(src.at[wid, pl.ds(bi * KB, KB)], sidx)
                pltpu.sync_copy(dst.at[wid, pl.ds(bi * KB, KB)], didx)
                g = [pltpu.async_copy(h.at[sidx.at[0]], rows[0], sems[0])]
                for j in range(KB):
                    if j + 1 < KB:
                        g.append(pltpu.async_copy(h.at[sidx.at[j + 1]],
                                                  rows[(j + 1) % 2],
                                                  sems[(j + 1) % 2]))
                    g[j].wait()
                    pltpu.sync_copy(rows[j % 2], acc_sh.at[didx.at[j]],
                                    add=True)
                return carry

            lax.fori_loop(0, k // KB, blk, 0)
        plsc.subcore_barrier()

        sl = pl.ds(s * rpt, rpt)
        pltpu.sync_copy(acc_sh.at[sl], accp.at[c, sl])

    return pl.kernel(
        body,
        out_type=out_type,
        mesh=mesh,
        scratch_types=scratch,
    )


def _tc_layer_body(relu, accp_ref, degp_ref, h_ref, wn_ref, wr_ref, b_ref,
                   out_ref):
    hh = h_ref[...]
    agg = accp_ref[0] + accp_ref[1] + hh
    deg = degp_ref[0, :, 0:1] + degp_ref[1, :, 0:1] + 1.0
    mean = agg / deg
    o = (jnp.dot(mean, wn_ref[...], preferred_element_type=jnp.float32)
         + jnp.dot(hh, wr_ref[...], preferred_element_type=jnp.float32)
         + b_ref[...])
    if relu:
        o = jnp.maximum(o, 0.0)
    out_ref[...] = o


def _tc_layer(accp, degp, h, wn, wr, b, relu, block):
    n, d = h.shape
    grid = n // block
    return pl.pallas_call(
        functools.partial(_tc_layer_body, relu),
        grid=(grid,),
        in_specs=[
            pl.BlockSpec((NC, block, d), lambda i: (0, i, 0)),
            pl.BlockSpec((NC, block, 1), lambda i: (0, i, 0)),
            pl.BlockSpec((block, d), lambda i: (i, 0)),
            pl.BlockSpec((d, d), lambda i: (0, 0)),
            pl.BlockSpec((d, d), lambda i: (0, 0)),
            pl.BlockSpec((1, d), lambda i: (0, 0)),
        ],
        out_specs=pl.BlockSpec((block, d), lambda i: (i, 0)),
        out_shape=jax.ShapeDtypeStruct((n, d), jnp.float32),
    )(accp, degp, h, wn, wr, b)


def kernel(x, edge_index, Wn0, Wr0, b0, Wn1, Wr1, b1, Wn2, Wr2, b2):
    n, d = x.shape
    e = edge_index.shape[1]
    epw = e // NW
    k = -(-epw // CHUNK)
    assert epw * NW == e and n % NS == 0

    rpt = (n // NS + CHUNK - 1) // CHUNK * CHUNK
    npad = rpt * NS
    src2 = edge_index[0].reshape(NW, epw)
    dst2 = edge_index[1].reshape(NW, epw)
    kb = -(-k // KB) * KB
    pad = kb * CHUNK - epw
    if pad:
        ap = jnp.arange(pad, dtype=jnp.int32)
        pad_src = jnp.broadcast_to((ap * 37) % n, (NW, pad))
        pad_dst = jnp.broadcast_to(n + ap % (npad - n), (NW, pad))
        src2 = jnp.concatenate([src2, pad_src], axis=1)
        dst2 = jnp.concatenate([dst2, pad_dst], axis=1)
    k = kb
    src = src2.reshape(NW, k, CHUNK)
    dst = dst2.reshape(NW, k, CHUNK)

    sc = _sc_aggregate(n, d, k, with_deg=False)
    scdeg = _sc_aggregate(n, d, k, with_deg=True)

    degp = scdeg(x, src, dst)[:, :, 0:1]
    accp = sc(x, src, dst)
    h1 = _tc_layer(accp, degp, x, Wn0, Wr0, b0.reshape(1, d), True, 2000)
    accp1 = sc(h1, src, dst)
    h2 = _tc_layer(accp1, degp, h1, Wn1, Wr1, b1.reshape(1, d), True, 2000)
    accp2 = sc(h2, src, dst)
    return _tc_layer(accp2, degp, h2, Wn2, Wr2, b2.reshape(1, d), False, 2000)


# speedup vs baseline: 14.4627x; 1.0008x over previous
import functools

import jax
import jax.numpy as jnp
from jax import lax
from jax.experimental import pallas as pl
from jax.experimental.pallas import tpu as pltpu
from jax.experimental.pallas import tpu_sc as plsc

NC = 2
NS = 16
NW = NC * NS

CHUNK = 128
KB = 8


def _sc_degree(n, dw, k):
    rpt = (n // NS + CHUNK - 1) // CHUNK * CHUNK
    npad = rpt * NS
    mesh = plsc.VectorSubcoreMesh(core_axis_name="c", subcore_axis_name="s",
                                  num_cores=NC, num_subcores=NS)
    scratch = [
        pltpu.VMEM((KB, CHUNK), jnp.int32),
        pltpu.VMEM((CHUNK, dw), jnp.float32),
        pltpu.VMEM_SHARED((npad, dw), jnp.float32),
        pltpu.SemaphoreType.DMA,
    ]

    def body(dst, degp, didx, ones, deg_sh, sem):
        c = lax.axis_index("c")
        s = lax.axis_index("s")
        wid = c * NS + s
        zv = jnp.zeros((16,), jnp.float32)
        ov = jnp.ones((16,), jnp.float32)

        def fill(val):
            def frow(i, carry):
                for t in range(dw // 16):
                    ones[i, pl.ds(t * 16, 16)] = val
                return carry
            lax.fori_loop(0, CHUNK, frow, 0)

        fill(zv)
        for j in range(rpt // CHUNK):
            pltpu.sync_copy(ones, deg_sh.at[pl.ds(s * rpt + j * CHUNK, CHUNK)])
        fill(ov)
        plsc.subcore_barrier()

        def dblk(bi, carry):
            pltpu.sync_copy(dst.at[wid, pl.ds(bi * KB, KB)], didx)
            ds_ = [pltpu.async_copy(ones, deg_sh.at[didx.at[j]], sem,
                                    add=True) for j in range(KB)]
            for dsc in ds_:
                dsc.wait()
            return carry

        lax.fori_loop(0, k // KB, dblk, 0)
        plsc.subcore_barrier()

        sl = pl.ds(s * rpt, rpt)
        pltpu.sync_copy(deg_sh.at[sl], degp.at[c, sl])

    return pl.kernel(
        body,
        out_type=jax.ShapeDtypeStruct((NC, npad, dw), jnp.float32),
        mesh=mesh,
        scratch_types=scratch,
    )


def _sc_aggregate(n, d, k):
    rpt = (n // NS + CHUNK - 1) // CHUNK * CHUNK
    npad = rpt * NS
    mesh = plsc.VectorSubcoreMesh(core_axis_name="c", subcore_axis_name="s",
                                  num_cores=NC, num_subcores=NS)

    out_type = jax.ShapeDtypeStruct((NC, npad, d), jnp.float32)
    scratch = [
        pltpu.VMEM((KB, CHUNK), jnp.int32),
        pltpu.VMEM((KB, CHUNK), jnp.int32),
        pltpu.VMEM((CHUNK, d), jnp.float32),
        pltpu.VMEM((CHUNK, d), jnp.float32),
        pltpu.VMEM_SHARED((npad, d), jnp.float32),
        pltpu.SemaphoreType.DMA,
        pltpu.SemaphoreType.DMA,
    ]

    def body(h, src, dst, accp, sidx, didx, rows0, rows1, acc_sh, s0, s1):
        c = lax.axis_index("c")
        s = lax.axis_index("s")
        wid = c * NS + s
        rows = (rows0, rows1)
        sems = (s0, s1)

        zv = jnp.zeros((16,), jnp.float32)

        def frow(i, carry):
            for t in range(d // 16):
                rows0[i, pl.ds(t * 16, 16)] = zv
            return carry

        lax.fori_loop(0, CHUNK, frow, 0)

        for j in range(rpt // CHUNK):
            sl = pl.ds(s * rpt + j * CHUNK, CHUNK)
            pltpu.sync_copy(rows0, acc_sh.at[sl])
        plsc.subcore_barrier()

        def blk(bi, carry):
            pltpu.sync_copy(src.at[wid, pl.ds(bi * KB, KB)], sidx)
            pltpu.sync_copy(dst.at[wid, pl.ds(bi * KB, KB)], didx)
            g = [pltpu.async_copy(h.at[sidx.at[0]], rows[0], sems[0])]
            for j in range(KB):
                if j + 1 < KB:
                    g.append(pltpu.async_copy(h.at[sidx.at[j + 1]],
                                              rows[(j + 1) % 2],
                                              sems[(j + 1) % 2]))
                g[j].wait()
                pltpu.sync_copy(rows[j % 2], acc_sh.at[didx.at[j]],
                                add=True)
            return carry

        lax.fori_loop(0, k // KB, blk, 0)
        plsc.subcore_barrier()

        sl = pl.ds(s * rpt, rpt)
        pltpu.sync_copy(acc_sh.at[sl], accp.at[c, sl])

    return pl.kernel(
        body,
        out_type=out_type,
        mesh=mesh,
        scratch_types=scratch,
    )


def _tc_layer_body(relu, accp_ref, degp_ref, h_ref, wn_ref, wr_ref, b_ref,
                   out_ref):
    hh = h_ref[...]
    agg = accp_ref[0] + accp_ref[1] + hh
    deg = degp_ref[0, :, 0:1] + degp_ref[1, :, 0:1] + 1.0
    mean = agg / deg
    o = (jnp.dot(mean, wn_ref[...], preferred_element_type=jnp.float32)
         + jnp.dot(hh, wr_ref[...], preferred_element_type=jnp.float32)
         + b_ref[...])
    if relu:
        o = jnp.maximum(o, 0.0)
    out_ref[...] = o


def _tc_layer(accp, degp, h, wn, wr, b, relu, block):
    n, d = h.shape
    grid = n // block
    return pl.pallas_call(
        functools.partial(_tc_layer_body, relu),
        grid=(grid,),
        in_specs=[
            pl.BlockSpec((NC, block, d), lambda i: (0, i, 0)),
            pl.BlockSpec((NC, block, 1), lambda i: (0, i, 0)),
            pl.BlockSpec((block, d), lambda i: (i, 0)),
            pl.BlockSpec((d, d), lambda i: (0, 0)),
            pl.BlockSpec((d, d), lambda i: (0, 0)),
            pl.BlockSpec((1, d), lambda i: (0, 0)),
        ],
        out_specs=pl.BlockSpec((block, d), lambda i: (i, 0)),
        out_shape=jax.ShapeDtypeStruct((n, d), jnp.float32),
    )(accp, degp, h, wn, wr, b)


def kernel(x, edge_index, Wn0, Wr0, b0, Wn1, Wr1, b1, Wn2, Wr2, b2):
    n, d = x.shape
    e = edge_index.shape[1]
    epw = e // NW
    k = -(-epw // CHUNK)
    assert epw * NW == e and n % NS == 0

    rpt = (n // NS + CHUNK - 1) // CHUNK * CHUNK
    npad = rpt * NS
    src2 = edge_index[0].reshape(NW, epw)
    dst2 = edge_index[1].reshape(NW, epw)
    kb = -(-k // KB) * KB
    pad = kb * CHUNK - epw
    if pad:
        ap = jnp.arange(pad, dtype=jnp.int32)
        pad_src = jnp.broadcast_to((ap * 37) % n, (NW, pad))
        pad_dst = jnp.broadcast_to(n + ap % (npad - n), (NW, pad))
        src2 = jnp.concatenate([src2, pad_src], axis=1)
        dst2 = jnp.concatenate([dst2, pad_dst], axis=1)
    k = kb
    src = src2.reshape(NW, k, CHUNK)
    dst = dst2.reshape(NW, k, CHUNK)

    sc = _sc_aggregate(n, d, k)
    scdeg = _sc_degree(n, d, k)

    degp = scdeg(dst)[:, :, 0:1]
    accp = sc(x, src, dst)
    h1 = _tc_layer(accp, degp, x, Wn0, Wr0, b0.reshape(1, d), True, 2000)
    accp1 = sc(h1, src, dst)
    h2 = _tc_layer(accp1, degp, h1, Wn1, Wr1, b1.reshape(1, d), True, 2000)
    accp2 = sc(h2, src, dst)
    return _tc_layer(accp2, degp, h2, Wn2, Wr2, b2.reshape(1, d), False, 2000)


# speedup vs baseline: 16.5003x; 1.1409x over previous
import functools

import jax
import jax.numpy as jnp
from jax import lax
from jax.experimental import pallas as pl
from jax.experimental.pallas import tpu as pltpu
from jax.experimental.pallas import tpu_sc as plsc

NC = 2
NS = 16
NW = NC * NS

CHUNK = 128
KB = 8


def _sc_degree(n, dw, k):
    rpt = (n // NS + CHUNK - 1) // CHUNK * CHUNK
    npad = rpt * NS
    mesh = plsc.VectorSubcoreMesh(core_axis_name="c", subcore_axis_name="s",
                                  num_cores=NC, num_subcores=NS)
    scratch = [
        pltpu.VMEM((KB, CHUNK), jnp.int32),
        pltpu.VMEM((CHUNK, dw), jnp.float32),
        pltpu.VMEM_SHARED((npad, dw), jnp.float32),
        pltpu.SemaphoreType.DMA,
    ]

    def body(dst, degp, didx, ones, deg_sh, sem):
        c = lax.axis_index("c")
        s = lax.axis_index("s")
        wid = c * NS + s
        zv = jnp.zeros((16,), jnp.float32)
        ov = jnp.ones((16,), jnp.float32)

        def fill(val):
            def frow(i, carry):
                for t in range(dw // 16):
                    ones[i, pl.ds(t * 16, 16)] = val
                return carry
            lax.fori_loop(0, CHUNK, frow, 0)

        fill(zv)
        for j in range(rpt // CHUNK):
            pltpu.sync_copy(ones, deg_sh.at[pl.ds(s * rpt + j * CHUNK, CHUNK)])
        fill(ov)
        plsc.subcore_barrier()

        def dblk(bi, carry):
            pltpu.sync_copy(dst.at[wid, pl.ds(bi * KB, KB)], didx)
            ds_ = [pltpu.async_copy(ones, deg_sh.at[didx.at[j]], sem,
                                    add=True) for j in range(KB)]
            for dsc in ds_:
                dsc.wait()
            return carry

        lax.fori_loop(0, k // KB, dblk, 0)
        plsc.subcore_barrier()

        sl = pl.ds(s * rpt, rpt)
        pltpu.sync_copy(deg_sh.at[sl], degp.at[c, sl])

    return pl.kernel(
        body,
        out_type=jax.ShapeDtypeStruct((NC, npad, dw), jnp.float32),
        mesh=mesh,
        scratch_types=scratch,
    )


def _sc_aggregate(n, d, k):
    rpt = (n // NS + CHUNK - 1) // CHUNK * CHUNK
    npad = rpt * NS
    mesh = plsc.VectorSubcoreMesh(core_axis_name="c", subcore_axis_name="s",
                                  num_cores=NC, num_subcores=NS)

    nblk = k // KB
    assert nblk % 2 == 0
    out_type = jax.ShapeDtypeStruct((NC, npad, d), jnp.float32)
    scratch = [
        pltpu.VMEM((KB, CHUNK), jnp.int32),
        pltpu.VMEM((KB, CHUNK), jnp.int32),
        pltpu.VMEM((KB, CHUNK), jnp.int32),
        pltpu.VMEM((KB, CHUNK), jnp.int32),
        pltpu.VMEM((CHUNK, d), jnp.float32),
        pltpu.VMEM((CHUNK, d), jnp.float32),
        pltpu.VMEM_SHARED((npad, d), jnp.float32),
        pltpu.SemaphoreType.DMA,
        pltpu.SemaphoreType.DMA,
        pltpu.SemaphoreType.DMA,
        pltpu.SemaphoreType.DMA,
    ]

    def body(h, src, dst, accp, sidxA, didxA, sidxB, didxB, rows0, rows1,
             acc_sh, s0, s1, is0, is1):
        c = lax.axis_index("c")
        s = lax.axis_index("s")
        wid = c * NS + s
        rows = (rows0, rows1)
        sems = (s0, s1)

        def pf_idx(bi, sx, dx):
            bc = jnp.minimum(bi, nblk - 1)
            a = pltpu.async_copy(src.at[wid, pl.ds(bc * KB, KB)], sx, is0)
            b = pltpu.async_copy(dst.at[wid, pl.ds(bc * KB, KB)], dx, is1)
            return a, b

        def pf_idx_wait(sx, dx):
            pltpu.make_async_copy(src.at[wid, pl.ds(0, KB)], sx, is0).wait()
            pltpu.make_async_copy(dst.at[wid, pl.ds(0, KB)], dx, is1).wait()

        zv = jnp.zeros((16,), jnp.float32)

        def frow(i, carry):
            for t in range(d // 16):
                rows0[i, pl.ds(t * 16, 16)] = zv
            return carry

        lax.fori_loop(0, CHUNK, frow, 0)

        for j in range(rpt // CHUNK):
            sl = pl.ds(s * rpt + j * CHUNK, CHUNK)
            pltpu.sync_copy(rows0, acc_sh.at[sl])
        plsc.subcore_barrier()

        pltpu.sync_copy(src.at[wid, pl.ds(0, KB)], sidxA)
        pltpu.sync_copy(dst.at[wid, pl.ds(0, KB)], didxA)
        pf_idx(1, sidxB, didxB)
        pltpu.async_copy(h.at[sidxA.at[0]], rows[0], sems[0])

        def run_block(bi, sx, dx, snx, dnx, g0):
            g = [g0]
            for j in range(KB):
                if j + 1 < KB:
                    g.append(pltpu.async_copy(h.at[sx.at[j + 1]],
                                              rows[(j + 1) % 2],
                                              sems[(j + 1) % 2]))
                else:
                    pf_idx_wait(snx, dnx)
                    g.append(pltpu.async_copy(h.at[snx.at[0]], rows[0],
                                              sems[0]))
                if g[j] is None:
                    pltpu.make_async_copy(h.at[sx.at[0]], rows[0],
                                          sems[0]).wait()
                else:
                    g[j].wait()
                pltpu.sync_copy(rows[j % 2], acc_sh.at[dx.at[j]], add=True)
            return g[KB]

        def superblk(sb, carry):
            b0 = sb * 2
            g1 = run_block(b0, sidxA, didxA, sidxB, didxB, None)
            pf_idx(b0 + 2, sidxA, didxA)
            run_block(b0 + 1, sidxB, didxB, sidxA, didxA, g1)
            pf_idx(b0 + 3, sidxB, didxB)
            return carry

        lax.fori_loop(0, nblk // 2, superblk, 0)
        pltpu.make_async_copy(h.at[sidxA.at[0]], rows[0], sems[0]).wait()
        pf_idx_wait(sidxB, didxB)
        plsc.subcore_barrier()

        sl = pl.ds(s * rpt, rpt)
        pltpu.sync_copy(acc_sh.at[sl], accp.at[c, sl])

    return pl.kernel(
        body,
        out_type=out_type,
        mesh=mesh,
        scratch_types=scratch,
    )


def _tc_layer_body(relu, accp_ref, degp_ref, h_ref, wn_ref, wr_ref, b_ref,
                   out_ref):
    hh = h_ref[...]
    agg = accp_ref[0] + accp_ref[1] + hh
    deg = degp_ref[0, :, 0:1] + degp_ref[1, :, 0:1] + 1.0
    mean = agg / deg
    o = (jnp.dot(mean, wn_ref[...], preferred_element_type=jnp.float32)
         + jnp.dot(hh, wr_ref[...], preferred_element_type=jnp.float32)
         + b_ref[...])
    if relu:
        o = jnp.maximum(o, 0.0)
    out_ref[...] = o


def _tc_layer(accp, degp, h, wn, wr, b, relu, block):
    n, d = h.shape
    grid = n // block
    return pl.pallas_call(
        functools.partial(_tc_layer_body, relu),
        grid=(grid,),
        in_specs=[
            pl.BlockSpec((NC, block, d), lambda i: (0, i, 0)),
            pl.BlockSpec((NC, block, 1), lambda i: (0, i, 0)),
            pl.BlockSpec((block, d), lambda i: (i, 0)),
            pl.BlockSpec((d, d), lambda i: (0, 0)),
            pl.BlockSpec((d, d), lambda i: (0, 0)),
            pl.BlockSpec((1, d), lambda i: (0, 0)),
        ],
        out_specs=pl.BlockSpec((block, d), lambda i: (i, 0)),
        out_shape=jax.ShapeDtypeStruct((n, d), jnp.float32),
    )(accp, degp, h, wn, wr, b)


def kernel(x, edge_index, Wn0, Wr0, b0, Wn1, Wr1, b1, Wn2, Wr2, b2):
    n, d = x.shape
    e = edge_index.shape[1]
    epw = e // NW
    k = -(-epw // CHUNK)
    assert epw * NW == e and n % NS == 0

    rpt = (n // NS + CHUNK - 1) // CHUNK * CHUNK
    npad = rpt * NS
    src2 = edge_index[0].reshape(NW, epw)
    dst2 = edge_index[1].reshape(NW, epw)
    kb = -(-k // KB) * KB
    pad = kb * CHUNK - epw
    if pad:
        ap = jnp.arange(pad, dtype=jnp.int32)
        pad_src = jnp.broadcast_to((ap * 37) % n, (NW, pad))
        pad_dst = jnp.broadcast_to(n + ap % (npad - n), (NW, pad))
        src2 = jnp.concatenate([src2, pad_src], axis=1)
        dst2 = jnp.concatenate([dst2, pad_dst], axis=1)
    k = kb
    src = src2.reshape(NW, k, CHUNK)
    dst = dst2.reshape(NW, k, CHUNK)

    sc = _sc_aggregate(n, d, k)
    scdeg = _sc_degree(n, d, k)

    degp = scdeg(dst)[:, :, 0:1]
    accp = sc(x, src, dst)
    h1 = _tc_layer(accp, degp, x, Wn0, Wr0, b0.reshape(1, d), True, 2000)
    accp1 = sc(h1, src, dst)
    h2 = _tc_layer(accp1, degp, h1, Wn1, Wr1, b1.reshape(1, d), True, 2000)
    accp2 = sc(h2, src, dst)
    return _tc_layer(accp2, degp, h2, Wn2, Wr2, b2.reshape(1, d), False, 2000)


# speedup vs baseline: 16.5257x; 1.0015x over previous
import functools

import jax
import jax.numpy as jnp
from jax import lax
from jax.experimental import pallas as pl
from jax.experimental.pallas import tpu as pltpu
from jax.experimental.pallas import tpu_sc as plsc

NC = 2
NS = 16
NW = NC * NS

CHUNK = 128
KB = 8


def _sc_degree(n, dw, k):
    rpt = (n // NS + CHUNK - 1) // CHUNK * CHUNK
    npad = rpt * NS
    mesh = plsc.VectorSubcoreMesh(core_axis_name="c", subcore_axis_name="s",
                                  num_cores=NC, num_subcores=NS)
    scratch = [
        pltpu.VMEM((KB, CHUNK), jnp.int32),
        pltpu.VMEM((CHUNK, dw), jnp.float32),
        pltpu.VMEM_SHARED((npad, dw), jnp.float32),
        pltpu.SemaphoreType.DMA,
    ]

    def body(dst, degp, didx, ones, deg_sh, sem):
        c = lax.axis_index("c")
        s = lax.axis_index("s")
        wid = c * NS + s
        zv = jnp.zeros((16,), jnp.float32)
        ov = jnp.ones((16,), jnp.float32)

        def fill(val):
            def frow(i, carry):
                for t in range(dw // 16):
                    ones[i, pl.ds(t * 16, 16)] = val
                return carry
            lax.fori_loop(0, CHUNK, frow, 0)

        fill(zv)
        for j in range(rpt // CHUNK):
            pltpu.sync_copy(ones, deg_sh.at[pl.ds(s * rpt + j * CHUNK, CHUNK)])
        fill(ov)
        plsc.subcore_barrier()

        def dblk(bi, carry):
            pltpu.sync_copy(dst.at[wid, pl.ds(bi * KB, KB)], didx)
            ds_ = [pltpu.async_copy(ones, deg_sh.at[didx.at[j]], sem,
                                    add=True) for j in range(KB)]
            for dsc in ds_:
                dsc.wait()
            return carry

        lax.fori_loop(0, k // KB, dblk, 0)
        plsc.subcore_barrier()

        sl = pl.ds(s * rpt, rpt)
        pltpu.sync_copy(deg_sh.at[sl], degp.at[c, sl])

    return pl.kernel(
        body,
        out_type=jax.ShapeDtypeStruct((NC, npad, dw), jnp.float32),
        mesh=mesh,
        scratch_types=scratch,
    )


def _sc_aggregate(n, d, k):
    rpt = (n // NS + CHUNK - 1) // CHUNK * CHUNK
    npad = rpt * NS
    mesh = plsc.VectorSubcoreMesh(core_axis_name="c", subcore_axis_name="s",
                                  num_cores=NC, num_subcores=NS)

    nblk = k // KB
    assert nblk % 2 == 0
    out_type = jax.ShapeDtypeStruct((NC, npad, d), jnp.float32)
    scratch = [
        pltpu.VMEM((KB, CHUNK), jnp.int32),
        pltpu.VMEM((KB, CHUNK), jnp.int32),
        pltpu.VMEM((KB, CHUNK), jnp.int32),
        pltpu.VMEM((KB, CHUNK), jnp.int32),
        pltpu.VMEM((CHUNK, d), jnp.float32),
        pltpu.VMEM((CHUNK, d), jnp.float32),
        pltpu.VMEM_SHARED((npad, d), jnp.float32),
        pltpu.SemaphoreType.DMA,
        pltpu.SemaphoreType.DMA,
        pltpu.SemaphoreType.DMA,
        pltpu.SemaphoreType.DMA,
        pltpu.SemaphoreType.DMA,
    ]

    def body(h, src, dst, accp, sidxA, didxA, sidxB, didxB, rows0, rows1,
             acc_sh, s0, s1, is0, is1, ssem):
        c = lax.axis_index("c")
        s = lax.axis_index("s")
        wid = c * NS + s
        rows = (rows0, rows1)
        sems = (s0, s1)

        def pf_idx(bi, sx, dx):
            bc = jnp.minimum(bi, nblk - 1)
            a = pltpu.async_copy(src.at[wid, pl.ds(bc * KB, KB)], sx, is0)
            b = pltpu.async_copy(dst.at[wid, pl.ds(bc * KB, KB)], dx, is1)
            return a, b

        def pf_idx_wait(sx, dx):
            pltpu.make_async_copy(src.at[wid, pl.ds(0, KB)], sx, is0).wait()
            pltpu.make_async_copy(dst.at[wid, pl.ds(0, KB)], dx, is1).wait()

        zv = jnp.zeros((16,), jnp.float32)

        def frow(i, carry):
            for t in range(d // 16):
                rows0[i, pl.ds(t * 16, 16)] = zv
            return carry

        lax.fori_loop(0, CHUNK, frow, 0)

        for j in range(rpt // CHUNK):
            sl = pl.ds(s * rpt + j * CHUNK, CHUNK)
            pltpu.sync_copy(rows0, acc_sh.at[sl])
        plsc.subcore_barrier()

        pltpu.sync_copy(src.at[wid, pl.ds(0, KB)], sidxA)
        pltpu.sync_copy(dst.at[wid, pl.ds(0, KB)], didxA)
        pf_idx(1, sidxB, didxB)
        pltpu.async_copy(h.at[sidxA.at[0]], rows[0], sems[0])

        def run_block(bi, sx, dx, snx, dnx, g0):
            g = [g0]
            scp = None
            for j in range(KB):
                if scp is not None:
                    scp.wait()
                if j + 1 < KB:
                    g.append(pltpu.async_copy(h.at[sx.at[j + 1]],
                                              rows[(j + 1) % 2],
                                              sems[(j + 1) % 2]))
                else:
                    pf_idx_wait(snx, dnx)
                    g.append(pltpu.async_copy(h.at[snx.at[0]], rows[0],
                                              sems[0]))
                if g[j] is None:
                    pltpu.make_async_copy(h.at[sx.at[0]], rows[0],
                                          sems[0]).wait()
                else:
                    g[j].wait()
                scp = pltpu.async_copy(rows[j % 2], acc_sh.at[dx.at[j]],
                                       ssem, add=True)
            return g[KB], scp

        def superblk(sb, carry):
            b0 = sb * 2
            g1, scp = run_block(b0, sidxA, didxA, sidxB, didxB, None)
            scp.wait()
            pf_idx(b0 + 2, sidxA, didxA)
            _, scp = run_block(b0 + 1, sidxB, didxB, sidxA, didxA, g1)
            scp.wait()
            pf_idx(b0 + 3, sidxB, didxB)
            return carry

        lax.fori_loop(0, nblk // 2, superblk, 0)
        pltpu.make_async_copy(h.at[sidxA.at[0]], rows[0], sems[0]).wait()
        pf_idx_wait(sidxB, didxB)
        plsc.subcore_barrier()

        sl = pl.ds(s * rpt, rpt)
        pltpu.sync_copy(acc_sh.at[sl], accp.at[c, sl])

    return pl.kernel(
        body,
        out_type=out_type,
        mesh=mesh,
        scratch_types=scratch,
    )


def _tc_layer_body(relu, accp_ref, degp_ref, h_ref, wn_ref, wr_ref, b_ref,
                   out_ref):
    hh = h_ref[...]
    agg = accp_ref[0] + accp_ref[1] + hh
    deg = degp_ref[0, :, 0:1] + degp_ref[1, :, 0:1] + 1.0
    mean = agg / deg
    o = (jnp.dot(mean, wn_ref[...], preferred_element_type=jnp.float32)
         + jnp.dot(hh, wr_ref[...], preferred_element_type=jnp.float32)
         + b_ref[...])
    if relu:
        o = jnp.maximum(o, 0.0)
    out_ref[...] = o


def _tc_layer(accp, degp, h, wn, wr, b, relu, block):
    n, d = h.shape
    grid = n // block
    return pl.pallas_call(
        functools.partial(_tc_layer_body, relu),
        grid=(grid,),
        in_specs=[
            pl.BlockSpec((NC, block, d), lambda i: (0, i, 0)),
            pl.BlockSpec((NC, block, 1), lambda i: (0, i, 0)),
            pl.BlockSpec((block, d), lambda i: (i, 0)),
            pl.BlockSpec((d, d), lambda i: (0, 0)),
            pl.BlockSpec((d, d), lambda i: (0, 0)),
            pl.BlockSpec((1, d), lambda i: (0, 0)),
        ],
        out_specs=pl.BlockSpec((block, d), lambda i: (i, 0)),
        out_shape=jax.ShapeDtypeStruct((n, d), jnp.float32),
    )(accp, degp, h, wn, wr, b)


def kernel(x, edge_index, Wn0, Wr0, b0, Wn1, Wr1, b1, Wn2, Wr2, b2):
    n, d = x.shape
    e = edge_index.shape[1]
    epw = e // NW
    k = -(-epw // CHUNK)
    assert epw * NW == e and n % NS == 0

    rpt = (n // NS + CHUNK - 1) // CHUNK * CHUNK
    npad = rpt * NS
    src2 = edge_index[0].reshape(NW, epw)
    dst2 = edge_index[1].reshape(NW, epw)
    kb = -(-k // KB) * KB
    pad = kb * CHUNK - epw
    if pad:
        ap = jnp.arange(pad, dtype=jnp.int32)
        pad_src = jnp.broadcast_to((ap * 37) % n, (NW, pad))
        pad_dst = jnp.broadcast_to(n + ap % (npad - n), (NW, pad))
        src2 = jnp.concatenate([src2, pad_src], axis=1)
        dst2 = jnp.concatenate([dst2, pad_dst], axis=1)
    k = kb
    src = src2.reshape(NW, k, CHUNK)
    dst = dst2.reshape(NW, k, CHUNK)

    sc = _sc_aggregate(n, d, k)
    scdeg = _sc_degree(n, d, k)

    degp = scdeg(dst)[:, :, 0:1]
    accp = sc(x, src, dst)
    h1 = _tc_layer(accp, degp, x, Wn0, Wr0, b0.reshape(1, d), True, 2000)
    accp1 = sc(h1, src, dst)
    h2 = _tc_layer(accp1, degp, h1, Wn1, Wr1, b1.reshape(1, d), True, 2000)
    accp2 = sc(h2, src, dst)
    return _tc_layer(accp2, degp, h2, Wn2, Wr2, b2.reshape(1, d), False, 2000)
